# Initial kernel scaffold; baseline (speedup 1.0000x reference)
import jax
import jax.numpy as jnp
from jax.experimental import pallas as pl


def kernel(x_RNA, x_ADT, sim_edge_index, sim_edge_weight, dist_edge_index, dist_edge_weight, common_edge_index, common_edge_weight, W1, b1, W2, b2, W3, b3, Ws, bs, Wd, bd, Wf1, bf1, Wf2, bf2):
    raise NotImplementedError("write your pallas kernel here")


# speedup vs baseline: 9.1941x; 9.1941x over previous
import functools

import jax
import jax.numpy as jnp
from jax import lax
from jax.experimental import pallas as pl
from jax.experimental.pallas import tpu as pltpu
from jax.experimental.pallas import tpu_sc as plsc

NC = 2
NS = 16
LANES = 16
K = 128


def _make_edge_scatter(S, E, N, D, use_table):
    assert E % K == 0
    NP = -(-N // (NS * 128)) * (NS * 128)
    CH = E // K
    RPT = NP // NS
    ZR = 128
    assert RPT % ZR == 0
    NW = NC * NS
    nch_base = CH // NW
    extra = CH - nch_base * NW

    mesh = plsc.VectorSubcoreMesh(
        core_axis_name="c", subcore_axis_name="s",
        num_cores=NC, num_subcores=NS)

    out_type = jax.ShapeDtypeStruct((S, NC, NP, D), jnp.float32)
    scratch = [
        pltpu.VMEM((K,), jnp.int32),
        pltpu.VMEM((K,), jnp.int32),
        pltpu.VMEM((K,), jnp.int32),
        pltpu.VMEM((K,), jnp.float32),
        pltpu.VMEM((K, D), jnp.float32),
        pltpu.VMEM((ZR, D), jnp.float32),
        pltpu.VMEM_SHARED((NP, D), jnp.float32),
        pltpu.SemaphoreType.DMA,
    ]

    def body(*refs):
        if use_table:
            (rows_h, cols_h, ws_h, table_h, out_h,
             colb, rowb, gidx, wb, valb, zb, acc, sem) = refs
        else:
            (rows_h, cols_h, ws_h, out_h,
             colb, rowb, gidx, wb, valb, zb, acc, sem) = refs
        c = lax.axis_index("c")
        s = lax.axis_index("s")
        wid = s * NC + c
        nch = jnp.where(wid < extra, nch_base + 1, nch_base)

        def zrow(r, carry):
            for f in range(D // LANES):
                zb[r, pl.ds(f * LANES, LANES)] = jnp.zeros((LANES,), jnp.float32)
            return carry
        lax.fori_loop(0, ZR, zrow, 0)

        for st in range(S):
            for z in range(RPT // ZR):
                pltpu.sync_copy(zb, acc.at[pl.ds(s * RPT + z * ZR, ZR)])
            plsc.subcore_barrier()

            def chunk(t, carry):
                base = st * E + (wid + t * NW) * K
                pltpu.sync_copy(cols_h.at[pl.ds(base, K)], colb)
                pltpu.sync_copy(ws_h.at[pl.ds(base, K)], wb)
                if use_table:
                    pltpu.sync_copy(rows_h.at[pl.ds(base, K)], rowb)
                    off = jnp.full((LANES,), st * N, jnp.int32)
                    def gx(j, cy):
                        gidx[pl.ds(j * LANES, LANES)] = (
                            rowb[pl.ds(j * LANES, LANES)] + off)
                        return cy
                    lax.fori_loop(0, K // LANES, gx, 0)
                    pltpu.async_copy(table_h.at[gidx], valb, sem).wait()
                    def scale(g, cy):
                        wv = wb[pl.ds(g * LANES, LANES)]
                        for i in range(LANES):
                            wsp = jnp.broadcast_to(wv[i], (LANES,))
                            e = g * LANES + i
                            for f in range(D // LANES):
                                sl = pl.ds(f * LANES, LANES)
                                valb[e, sl] = valb[e, sl] * wsp
                        return cy
                    lax.fori_loop(0, K // LANES, scale, 0)
                else:
                    def scale(g, cy):
                        wv = wb[pl.ds(g * LANES, LANES)]
                        for i in range(LANES):
                            wsp = jnp.broadcast_to(wv[i], (LANES,))
                            e = g * LANES + i
                            for f in range(D // LANES):
                                valb[e, pl.ds(f * LANES, LANES)] = wsp
                        return cy
                    lax.fori_loop(0, K // LANES, scale, 0)
                pltpu.sync_copy(valb, acc.at[colb], add=True)
                return carry
            lax.fori_loop(0, nch, chunk, 0)
            plsc.subcore_barrier()

            for z in range(RPT // ZR):
                r0 = s * RPT + z * ZR
                pltpu.sync_copy(acc.at[pl.ds(r0, ZR)],
                                out_h.at[st, c, pl.ds(r0, ZR)])
            plsc.subcore_barrier()

    return functools.partial(
        pl.kernel, out_type=out_type, mesh=mesh, scratch_types=scratch)(body)


_R = 1024


def _dinv_tc(degs):
    S, _, n, _ = degs.shape

    def body(d_ref, o_ref):
        d = 1.0 + d_ref[:, 0, :, 0] + d_ref[:, 1, :, 0]
        o_ref[...] = jnp.where(d > 0, lax.rsqrt(jnp.where(d > 0, d, 1.0)), 0.0)

    return pl.pallas_call(
        body,
        out_shape=jax.ShapeDtypeStruct((S, n), jnp.float32),
    )(degs)


def _matmul3_tc(x1, x2, w1, w2, w3, dinv):
    n = x1.shape[0]
    D = w1.shape[1]
    grid = -(-n // _R)

    def body(x1_ref, x2_ref, w1_ref, w2_ref, w3_ref, dv_ref, o_ref):
        i = pl.program_id(0)
        h1 = jnp.dot(x1_ref[...], w1_ref[...],
                     preferred_element_type=jnp.float32)
        h2 = jnp.dot(x1_ref[...], w2_ref[...],
                     preferred_element_type=jnp.float32)
        h3 = jnp.dot(x2_ref[...], w3_ref[...],
                     preferred_element_type=jnp.float32)
        o_ref[0] = h1 * dv_ref[0, pl.ds(i * _R, _R)][:, None]
        o_ref[1] = h2 * dv_ref[1, pl.ds(i * _R, _R)][:, None]
        o_ref[2] = h3 * dv_ref[2, pl.ds(i * _R, _R)][:, None]

    return pl.pallas_call(
        body,
        grid=(grid,),
        in_specs=[
            pl.BlockSpec((_R, x1.shape[1]), lambda i: (i, 0)),
            pl.BlockSpec((_R, x2.shape[1]), lambda i: (i, 0)),
            pl.BlockSpec(w1.shape, lambda i: (0, 0)),
            pl.BlockSpec(w2.shape, lambda i: (0, 0)),
            pl.BlockSpec(w3.shape, lambda i: (0, 0)),
            pl.BlockSpec(dinv.shape, lambda i: (0, 0)),
        ],
        out_specs=pl.BlockSpec((3, _R, D), lambda i: (0, i, 0)),
        out_shape=jax.ShapeDtypeStruct((3, n, D), jnp.float32),
    )(x1, x2, w1, w2, w3, dinv)


def _mid_tc(acc1, h123, dinv, ws, wd, b1, b2, b3):
    _, n, D = h123.shape
    grid = -(-n // _R)

    def body(a_ref, h_ref, dv_ref, ws_ref, wd_ref, b1_ref, b2_ref, b3_ref,
             pro_ref, hw_ref):
        i = pl.program_id(0)
        dv0 = dv_ref[0, pl.ds(i * _R, _R)][:, None]
        dv1 = dv_ref[1, pl.ds(i * _R, _R)][:, None]
        dv2 = dv_ref[2, pl.ds(i * _R, _R)][:, None]
        a0 = a_ref[0, 0] + a_ref[0, 1] + h_ref[0]
        a1 = a_ref[1, 0] + a_ref[1, 1] + h_ref[1]
        a2 = a_ref[2, 0] + a_ref[2, 1] + h_ref[2]
        xs = jax.nn.relu(a0 * dv0 + b1_ref[...][None, :])
        xd = jax.nn.relu(a1 * dv1 + b2_ref[...][None, :])
        pro_ref[...] = a2 * dv2 + b3_ref[...][None, :]
        hs = jnp.dot(xs, ws_ref[...], preferred_element_type=jnp.float32)
        hd = jnp.dot(xd, wd_ref[...], preferred_element_type=jnp.float32)
        hw_ref[0] = hs * dv0
        hw_ref[1] = hd * dv1

    return pl.pallas_call(
        body,
        grid=(grid,),
        in_specs=[
            pl.BlockSpec((3, NC, _R, D), lambda i: (0, 0, i, 0)),
            pl.BlockSpec((3, _R, D), lambda i: (0, i, 0)),
            pl.BlockSpec(dinv.shape, lambda i: (0, 0)),
            pl.BlockSpec(ws.shape, lambda i: (0, 0)),
            pl.BlockSpec(wd.shape, lambda i: (0, 0)),
            pl.BlockSpec(b1.shape, lambda i: (0,)),
            pl.BlockSpec(b2.shape, lambda i: (0,)),
            pl.BlockSpec(b3.shape, lambda i: (0,)),
        ],
        out_specs=[
            pl.BlockSpec((_R, D), lambda i: (i, 0)),
            pl.BlockSpec((2, _R, D), lambda i: (0, i, 0)),
        ],
        out_shape=[
            jax.ShapeDtypeStruct((n, D), jnp.float32),
            jax.ShapeDtypeStruct((2, n, D), jnp.float32),
        ],
    )(acc1, h123, dinv, ws, wd, b1, b2, b3)


def _final_tc(acc2, hw2, dinv, bs, bd, pro, wf1a, wf1b, wf2a, wf2b, bf1, bf2):
    _, n, D = hw2.shape
    grid = -(-n // _R)

    def body(a_ref, h_ref, dv_ref, bs_ref, bd_ref, pro_ref,
             w1a_ref, w1b_ref, w2a_ref, w2b_ref, bf1_ref, bf2_ref,
             xs_ref, xd_ref, f_ref, fp_ref):
        i = pl.program_id(0)
        a0 = a_ref[0, 0] + a_ref[0, 1] + h_ref[0]
        a1 = a_ref[1, 0] + a_ref[1, 1] + h_ref[1]
        x_sim = a0 * dv_ref[0, pl.ds(i * _R, _R)][:, None] + bs_ref[...][None, :]
        x_dist = a1 * dv_ref[1, pl.ds(i * _R, _R)][:, None] + bd_ref[...][None, :]
        fused = (jnp.dot(x_sim, w1a_ref[...], preferred_element_type=jnp.float32)
                 + jnp.dot(x_dist, w1b_ref[...], preferred_element_type=jnp.float32)
                 + bf1_ref[...][None, :])
        fp = (jnp.dot(fused, w2a_ref[...], preferred_element_type=jnp.float32)
              + jnp.dot(pro_ref[...], w2b_ref[...], preferred_element_type=jnp.float32)
              + bf2_ref[...][None, :])
        xs_ref[...] = x_sim
        xd_ref[...] = x_dist
        f_ref[...] = fused
        fp_ref[...] = fp

    os = jax.ShapeDtypeStruct((n, D), jnp.float32)
    return pl.pallas_call(
        body,
        grid=(grid,),
        in_specs=[
            pl.BlockSpec((2, NC, _R, D), lambda i: (0, 0, i, 0)),
            pl.BlockSpec((2, _R, D), lambda i: (0, i, 0)),
            pl.BlockSpec(dinv.shape, lambda i: (0, 0)),
            pl.BlockSpec(bs.shape, lambda i: (0,)),
            pl.BlockSpec(bd.shape, lambda i: (0,)),
            pl.BlockSpec((_R, D), lambda i: (i, 0)),
            pl.BlockSpec(wf1a.shape, lambda i: (0, 0)),
            pl.BlockSpec(wf1b.shape, lambda i: (0, 0)),
            pl.BlockSpec(wf2a.shape, lambda i: (0, 0)),
            pl.BlockSpec(wf2b.shape, lambda i: (0, 0)),
            pl.BlockSpec(bf1.shape, lambda i: (0,)),
            pl.BlockSpec(bf2.shape, lambda i: (0,)),
        ],
        out_specs=[pl.BlockSpec((_R, D), lambda i: (i, 0))] * 4,
        out_shape=[os, os, os, os],
    )(acc2, hw2, dinv, bs, bd, pro, wf1a, wf1b, wf2a, wf2b, bf1, bf2)


def kernel(x_RNA, x_ADT, sim_edge_index, sim_edge_weight, dist_edge_index,
           dist_edge_weight, common_edge_index, common_edge_weight,
           W1, b1, W2, b2, W3, b3, Ws, bs, Wd, bd, Wf1, bf1, Wf2, bf2):
    n, D = x_RNA.shape[0], W1.shape[1]
    E = sim_edge_weight.shape[0]

    rows3 = jnp.stack([sim_edge_index[0], dist_edge_index[0],
                       common_edge_index[0]])
    cols3 = jnp.stack([sim_edge_index[1], dist_edge_index[1],
                       common_edge_index[1]])
    ws3 = jnp.stack([sim_edge_weight, dist_edge_weight, common_edge_weight])

    rows3f, cols3f, ws3f = (rows3.reshape(-1), cols3.reshape(-1),
                            ws3.reshape(-1))

    deg_fn = _make_edge_scatter(3, E, n, D, use_table=False)
    degs = deg_fn(rows3f, cols3f, ws3f)
    dinv = _dinv_tc(degs)

    h123 = _matmul3_tc(x_RNA, x_ADT, W1, W2, W3, dinv)

    prop3_fn = _make_edge_scatter(3, E, n, D, use_table=True)
    acc1 = prop3_fn(rows3f, cols3f, ws3f, h123.reshape(3 * n, D))

    pro, hw2 = _mid_tc(acc1, h123, dinv, Ws, Wd, b1, b2, b3)

    prop2_fn = _make_edge_scatter(2, E, n, D, use_table=True)
    acc2 = prop2_fn(rows3f[:2 * E], cols3f[:2 * E], ws3f[:2 * E],
                    hw2.reshape(2 * n, D))

    x_sim, x_dist, fused, fused_pro = _final_tc(
        acc2, hw2, dinv, bs, bd, pro,
        Wf1[:D], Wf1[D:], Wf2[:D], Wf2[D:], bf1, bf2)

    return (x_sim, x_dist, fused, fused_pro, pro)
